# Initial kernel scaffold; baseline (speedup 1.0000x reference)
import functools

import jax
import jax.numpy as jnp
from jax import lax
from jax.experimental import pallas as pl
from jax.experimental.pallas import tpu as pltpu
from jax.experimental.pallas import tpu_sc as plsc

N = 10000
E = 320000
D = 128

NC = 2
NS = 16
L = 16
EC = E // NC
EW = EC // NS
CHUNK = 80
NWIN = EW // CHUNK
RPT = N // NS
ZROWS = 125

_mesh = plsc.VectorSubcoreMesh(core_axis_name="c", subcore_axis_name="s")


@functools.partial(
    pl.kernel,
    out_type=jax.ShapeDtypeStruct((NC, N, D), jnp.float32),
    mesh=_mesh,
    scratch_types=[
        pltpu.VMEM((CHUNK,), jnp.int32),
        pltpu.VMEM((CHUNK,), jnp.int32),
        pltpu.VMEM((CHUNK,), jnp.float32),
        pltpu.VMEM((CHUNK, D), jnp.float32),
        pltpu.VMEM((ZROWS, D), jnp.float32),
        pltpu.VMEM_SHARED((N, D), jnp.float32),
        pltpu.SemaphoreType.DMA,
    ],
)
def _sc_aggregate(x_hbm, src_hbm, dst_hbm, w_hbm, out_hbm,
                  src_v, dst_v, w_v, rows_v, zb_v, acc_sh, sem):
    c = lax.axis_index("c")
    s = lax.axis_index("s")

    def zrow(i, carry):
        for j in range(D // L):
            zb_v[i, pl.ds(j * L, L)] = jnp.zeros((L,), jnp.float32)
        return carry

    lax.fori_loop(0, ZROWS, zrow, 0)
    for t in range(RPT // ZROWS):
        pltpu.sync_copy(zb_v, acc_sh.at[pl.ds(s * RPT + t * ZROWS, ZROWS)])
    plsc.subcore_barrier()

    base = c * EC + s * EW

    def window(i, carry):
        off = base + i * CHUNK
        pltpu.sync_copy(src_hbm.at[pl.ds(off, CHUNK)], src_v)
        pltpu.sync_copy(dst_hbm.at[pl.ds(off, CHUNK)], dst_v)
        pltpu.sync_copy(w_hbm.at[pl.ds(off, CHUNK)], w_v)
        pltpu.async_copy(x_hbm.at[src_v], rows_v, sem).wait()
        for g in range(CHUNK // L):
            w16 = w_v[pl.ds(g * L, L)]

            def lane(k, carry2):
                wb = w16.at[jnp.full((L,), k, jnp.int32)].get(
                    mode="promise_in_bounds")
                r = g * L + k
                for j in range(D // L):
                    rows_v[r, pl.ds(j * L, L)] = rows_v[r, pl.ds(j * L, L)] * wb
                return carry2

            lax.fori_loop(0, L, lane, 0)
        pltpu.sync_copy(rows_v, acc_sh.at[dst_v], add=True)
        return carry

    lax.fori_loop(0, NWIN, window, 0)
    plsc.subcore_barrier()
    pltpu.sync_copy(acc_sh.at[pl.ds(s * RPT, RPT)],
                    out_hbm.at[c, pl.ds(s * RPT, RPT)])


def _tc_body(p_ref, w_ref, o_ref):
    agg = p_ref[0] + p_ref[1]
    pre = jnp.dot(agg, w_ref[...], preferred_element_type=jnp.float32)
    mean = jnp.mean(pre, axis=0, keepdims=True)
    var = jnp.mean(pre * pre, axis=0, keepdims=True) - mean * mean
    o_ref[...] = jnp.maximum((pre - mean) * lax.rsqrt(var + 0.001), 0.0)


def kernel(x, edge_index, edge_weight, W):
    partials = _sc_aggregate(x, edge_index[0], edge_index[1], edge_weight)
    return pl.pallas_call(
        _tc_body,
        out_shape=jax.ShapeDtypeStruct((N, D), jnp.float32),
    )(partials, W)


# speedup vs baseline: 4.2537x; 4.2537x over previous
import functools

import jax
import jax.numpy as jnp
from jax import lax
from jax.experimental import pallas as pl
from jax.experimental.pallas import tpu as pltpu
from jax.experimental.pallas import tpu_sc as plsc

N = 10000
E = 320000
D = 128

NC = 2
NS = 16
L = 16
EC = E // NC
EW = EC // NS
CHUNK = 80
NWIN = EW // CHUNK
NP = 10240
RPT = NP // NS
ZROWS = 128

_mesh = plsc.VectorSubcoreMesh(core_axis_name="c", subcore_axis_name="s")


@functools.partial(
    pl.kernel,
    out_type=jax.ShapeDtypeStruct((NC, NP, D), jnp.float32),
    mesh=_mesh,
    scratch_types=[
        pltpu.VMEM((CHUNK,), jnp.int32),
        pltpu.VMEM((CHUNK,), jnp.int32),
        pltpu.VMEM((CHUNK,), jnp.float32),
        pltpu.VMEM((CHUNK, D), jnp.float32),
        pltpu.VMEM((ZROWS, D), jnp.float32),
        pltpu.VMEM_SHARED((NP, D), jnp.float32),
        pltpu.SemaphoreType.DMA,
    ],
)
def _sc_aggregate(x_hbm, src_hbm, dst_hbm, w_hbm, out_hbm,
                  src_v, dst_v, w_v, rows_v, zb_v, acc_sh, sem):
    c = lax.axis_index("c")
    s = lax.axis_index("s")

    def zrow(i, carry):
        for j in range(D // L):
            zb_v[i, pl.ds(j * L, L)] = jnp.zeros((L,), jnp.float32)
        return carry

    lax.fori_loop(0, ZROWS, zrow, 0)
    for t in range(RPT // ZROWS):
        pltpu.sync_copy(zb_v, acc_sh.at[pl.ds(s * RPT + t * ZROWS, ZROWS)])
    plsc.subcore_barrier()

    base = c * EC + s * EW

    def window(i, carry):
        off = base + i * CHUNK
        pltpu.sync_copy(src_hbm.at[pl.ds(off, CHUNK)], src_v)
        pltpu.sync_copy(dst_hbm.at[pl.ds(off, CHUNK)], dst_v)
        pltpu.sync_copy(w_hbm.at[pl.ds(off, CHUNK)], w_v)
        pltpu.async_copy(x_hbm.at[src_v], rows_v, sem).wait()
        for g in range(CHUNK // L):
            w16 = w_v[pl.ds(g * L, L)]

            def lane(k, carry2):
                wb = w16.at[jnp.full((L,), k, jnp.int32)].get(
                    mode="promise_in_bounds")
                r = g * L + k
                for j in range(D // L):
                    rows_v[r, pl.ds(j * L, L)] = rows_v[r, pl.ds(j * L, L)] * wb
                return carry2

            lax.fori_loop(0, L, lane, 0)
        pltpu.sync_copy(rows_v, acc_sh.at[dst_v], add=True)
        return carry

    lax.fori_loop(0, NWIN, window, 0)
    plsc.subcore_barrier()
    pltpu.sync_copy(acc_sh.at[pl.ds(s * RPT, RPT)],
                    out_hbm.at[c, pl.ds(s * RPT, RPT)])


def _tc_body(p_ref, w_ref, o_ref):
    agg = p_ref[0, :N, :] + p_ref[1, :N, :]
    pre = jnp.dot(agg, w_ref[...], preferred_element_type=jnp.float32)
    mean = jnp.mean(pre, axis=0, keepdims=True)
    var = jnp.mean(pre * pre, axis=0, keepdims=True) - mean * mean
    o_ref[...] = jnp.maximum((pre - mean) * lax.rsqrt(var + 0.001), 0.0)


def kernel(x, edge_index, edge_weight, W):
    partials = _sc_aggregate(x, edge_index[0], edge_index[1], edge_weight)
    return pl.pallas_call(
        _tc_body,
        out_shape=jax.ShapeDtypeStruct((N, D), jnp.float32),
    )(partials, W)


# speedup vs baseline: 9.0460x; 2.1266x over previous
import functools

import jax
import jax.numpy as jnp
from jax import lax
from jax.experimental import pallas as pl
from jax.experimental.pallas import tpu as pltpu
from jax.experimental.pallas import tpu_sc as plsc

N = 10000
E = 320000
D = 128

NC = 2
NS = 16
L = 16
EP = 327680
EC = EP // NC
EW = EC // NS
CHUNK = 64
NB = 4
GE = NB * CHUNK
NG = EW // GE
NP = 10240
RPT = NP // NS
ZROWS = 40

_mesh = plsc.VectorSubcoreMesh(core_axis_name="c", subcore_axis_name="s")


@functools.partial(
    pl.kernel,
    out_type=jax.ShapeDtypeStruct((NC, NP, D), jnp.float32),
    mesh=_mesh,
    scratch_types=[
        pltpu.VMEM((2, GE), jnp.int32),
        pltpu.VMEM((2, GE), jnp.float32),
        pltpu.VMEM((2, NB, CHUNK), jnp.int32),
        pltpu.VMEM((NB, CHUNK, D), jnp.float32),
        pltpu.VMEM((ZROWS, D), jnp.float32),
        pltpu.VMEM_SHARED((NP, D), jnp.float32),
        pltpu.SemaphoreType.DMA,
        pltpu.SemaphoreType.DMA((NB,)),
        pltpu.SemaphoreType.DMA((NB,)),
    ],
)
def _sc_aggregate(x_hbm, src_hbm, dst_hbm, w_hbm, out_hbm,
                  src_v, w_v, dst_v, rows_v, zb_v, acc_sh,
                  sem_i, sem_g, sem_s):
    c = lax.axis_index("c")
    s = lax.axis_index("s")
    base = c * EC + s * EW

    def idx_copies(g, p):
        off = base + g * GE
        cps = [
            (src_hbm.at[pl.ds(off, GE)], src_v.at[p]),
            (w_hbm.at[pl.ds(off, GE)], w_v.at[p]),
        ]
        for b in range(NB):
            cps.append((dst_hbm.at[pl.ds(off + b * CHUNK, CHUNK)],
                        dst_v.at[p, b]))
        return cps

    for src, dst in idx_copies(0, 0):
        pltpu.async_copy(src, dst, sem_i)

    def zrow(i, carry):
        for j in range(D // L):
            zb_v[i, pl.ds(j * L, L)] = jnp.zeros((L,), jnp.float32)
        return carry

    lax.fori_loop(0, ZROWS, zrow, 0)
    for t in range(RPT // ZROWS):
        pltpu.sync_copy(zb_v, acc_sh.at[pl.ds(s * RPT + t * ZROWS, ZROWS)])
    plsc.subcore_barrier()

    def group(g, p):
        for src, dst in idx_copies(g, p):
            pltpu.make_async_copy(src, dst, sem_i).wait()
        gathers = []
        for b in range(NB):
            gathers.append(pltpu.async_copy(
                x_hbm.at[src_v.at[p, pl.ds(b * CHUNK, CHUNK)]],
                rows_v.at[b], sem_g.at[b]))
        gnext = jnp.minimum(g + 1, NG - 1)
        for src, dst in idx_copies(gnext, 1 - p):
            pltpu.async_copy(src, dst, sem_i)
        scatters = []
        for b in range(NB):
            gathers[b].wait()

            def blk(t, carry):
                w16 = w_v[p, pl.ds(b * CHUNK + t * L, L)]
                for k in range(L):
                    wb = w16.at[jnp.full((L,), k, jnp.int32)].get(
                        mode="promise_in_bounds")
                    r = t * L + k
                    for j in range(D // L):
                        rows_v[b, r, pl.ds(j * L, L)] = (
                            rows_v[b, r, pl.ds(j * L, L)] * wb)
                return carry

            lax.fori_loop(0, CHUNK // L, blk, 0)
            scatters.append(pltpu.async_copy(
                rows_v.at[b], acc_sh.at[dst_v.at[p, b]], sem_s.at[b],
                add=True))
        for h in scatters:
            h.wait()
        return ()

    def pair(i, carry):
        group(2 * i, 0)
        group(2 * i + 1, 1)
        return carry

    lax.fori_loop(0, NG // 2, pair, 0)
    for src, dst in idx_copies(NG - 1, 0):
        pltpu.make_async_copy(src, dst, sem_i).wait()

    plsc.subcore_barrier()
    pltpu.sync_copy(acc_sh.at[pl.ds(s * RPT, RPT)],
                    out_hbm.at[c, pl.ds(s * RPT, RPT)])


def _tc_body(p_ref, w_ref, o_ref):
    agg = p_ref[0, :N, :] + p_ref[1, :N, :]
    pre = jnp.dot(agg, w_ref[...], preferred_element_type=jnp.float32)
    mean = jnp.mean(pre, axis=0, keepdims=True)
    var = jnp.mean(pre * pre, axis=0, keepdims=True) - mean * mean
    o_ref[...] = jnp.maximum((pre - mean) * lax.rsqrt(var + 0.001), 0.0)


def kernel(x, edge_index, edge_weight, W):
    npad = EP - E
    pad_nodes = (jnp.arange(npad, dtype=jnp.int32) * 53) % N
    src = jnp.concatenate([edge_index[0], pad_nodes])
    dst = jnp.concatenate([edge_index[1], pad_nodes])
    w = jnp.concatenate([edge_weight, jnp.zeros((npad,), jnp.float32)])
    partials = _sc_aggregate(x, src, dst, w)
    return pl.pallas_call(
        _tc_body,
        out_shape=jax.ShapeDtypeStruct((N, D), jnp.float32),
    )(partials, W)


# speedup vs baseline: 10.5701x; 1.1685x over previous
import functools

import jax
import jax.numpy as jnp
from jax import lax
from jax.experimental import pallas as pl
from jax.experimental.pallas import tpu as pltpu
from jax.experimental.pallas import tpu_sc as plsc

N = 10000
E = 320000
D = 128

NC = 2
NS = 16
L = 16
CHUNK = 128
NB = 2
GE = NB * CHUNK
EW = 10240
NGF = EW // GE
NGL = 10
NP = 10240
RPT = NP // NS
ZROWS = 40

_mesh = plsc.VectorSubcoreMesh(core_axis_name="c", subcore_axis_name="s")


@functools.partial(
    pl.kernel,
    out_type=jax.ShapeDtypeStruct((NC, NP, D), jnp.float32),
    mesh=_mesh,
    scratch_types=[
        pltpu.VMEM((2, GE), jnp.int32),
        pltpu.VMEM((2, GE), jnp.float32),
        pltpu.VMEM((2, NB, CHUNK), jnp.int32),
        pltpu.VMEM((NB, CHUNK, D), jnp.float32),
        pltpu.VMEM((ZROWS, D), jnp.float32),
        pltpu.VMEM_SHARED((NP, D), jnp.float32),
        pltpu.SemaphoreType.DMA,
        pltpu.SemaphoreType.DMA((NB,)),
        pltpu.SemaphoreType.DMA((NB,)),
    ],
)
def _sc_aggregate(x_hbm, ei_hbm, w_hbm, out_hbm,
                  src_v, w_v, dst_v, rows_v, zb_v, acc_sh,
                  sem_i, sem_g, sem_s):
    c = lax.axis_index("c")
    s = lax.axis_index("s")
    wid = c * NS + s
    base = wid * EW
    ng = jnp.where(wid == NC * NS - 1, NGL, NGF)

    def idx_copies(g, p):
        off = base + g * GE
        cps = [
            (ei_hbm.at[0, pl.ds(off, GE)], src_v.at[p]),
            (w_hbm.at[pl.ds(off, GE)], w_v.at[p]),
        ]
        for b in range(NB):
            cps.append((ei_hbm.at[1, pl.ds(off + b * CHUNK, CHUNK)],
                        dst_v.at[p, b]))
        return cps

    def scatter_copy(b, p):
        return (rows_v.at[b], acc_sh.at[dst_v.at[p, b]])

    for src, dst in idx_copies(0, 0):
        pltpu.async_copy(src, dst, sem_i)

    def zrow(i, carry):
        for j in range(D // L):
            zb_v[i, pl.ds(j * L, L)] = jnp.zeros((L,), jnp.float32)
        return carry

    lax.fori_loop(0, ZROWS, zrow, 0)
    for t in range(RPT // ZROWS):
        pltpu.sync_copy(zb_v, acc_sh.at[pl.ds(s * RPT + t * ZROWS, ZROWS)])
    plsc.subcore_barrier()

    def group(g, p, drain):
        for src, dst in idx_copies(g, p):
            pltpu.make_async_copy(src, dst, sem_i).wait()
        for b in range(NB):
            if drain:
                sc_src, sc_dst = scatter_copy(b, 1 - p)
                pltpu.make_async_copy(sc_src, sc_dst, sem_s.at[b]).wait()
            pltpu.async_copy(
                x_hbm.at[src_v.at[p, pl.ds(b * CHUNK, CHUNK)]],
                rows_v.at[b], sem_g.at[b])
        gnext = jnp.minimum(g + 1, ng - 1)
        for src, dst in idx_copies(gnext, 1 - p):
            pltpu.async_copy(src, dst, sem_i)
        for b in range(NB):
            pltpu.make_async_copy(
                x_hbm.at[src_v.at[p, pl.ds(b * CHUNK, CHUNK)]],
                rows_v.at[b], sem_g.at[b]).wait()

            def blk(t, carry):
                w16 = w_v[p, pl.ds(b * CHUNK + t * L, L)]
                for k in range(L):
                    wb = w16.at[jnp.full((L,), k, jnp.int32)].get(
                        mode="promise_in_bounds")
                    r = t * L + k
                    for j in range(D // L):
                        rows_v[b, r, pl.ds(j * L, L)] = (
                            rows_v[b, r, pl.ds(j * L, L)] * wb)
                return carry

            lax.fori_loop(0, CHUNK // L, blk, 0)
            sc_src, sc_dst = scatter_copy(b, p)
            pltpu.async_copy(sc_src, sc_dst, sem_s.at[b], add=True)

    group(0, 0, drain=False)
    group(1, 1, drain=True)

    def pair(i, carry):
        group(2 * i, 0, drain=True)
        group(2 * i + 1, 1, drain=True)
        return carry

    lax.fori_loop(1, ng // 2, pair, 0)
    for src, dst in idx_copies(ng - 1, 0):
        pltpu.make_async_copy(src, dst, sem_i).wait()
    for b in range(NB):
        sc_src, sc_dst = scatter_copy(b, 1)
        pltpu.make_async_copy(sc_src, sc_dst, sem_s.at[b]).wait()

    plsc.subcore_barrier()
    pltpu.sync_copy(acc_sh.at[pl.ds(s * RPT, RPT)],
                    out_hbm.at[c, pl.ds(s * RPT, RPT)])


def _tc_body(p_ref, w_ref, o_ref):
    agg = p_ref[0, :N, :] + p_ref[1, :N, :]
    pre = jnp.dot(agg, w_ref[...], preferred_element_type=jnp.float32)
    mean = jnp.mean(pre, axis=0, keepdims=True)
    var = jnp.mean(pre * pre, axis=0, keepdims=True) - mean * mean
    o_ref[...] = jnp.maximum((pre - mean) * lax.rsqrt(var + 0.001), 0.0)


def kernel(x, edge_index, edge_weight, W):
    partials = _sc_aggregate(x, edge_index, edge_weight)
    return pl.pallas_call(
        _tc_body,
        out_shape=jax.ShapeDtypeStruct((N, D), jnp.float32),
    )(partials, W)
